# Initial kernel scaffold; baseline (speedup 1.0000x reference)
import jax
import jax.numpy as jnp
from jax.experimental import pallas as pl


def kernel(x, node_labels, edge_index, edge_attr, batch, x2, roi_labels, edge_index2, edge_attr2, batch2, W1, b1, W2, b2, Wr1, br1, Wr2, br2, mW1, mb1, bn_gamma, bn_beta, mW2, mb2):
    raise NotImplementedError("write your pallas kernel here")


# speedup vs baseline: 5.3303x; 5.3303x over previous
import functools
import math

import jax
import jax.numpy as jnp
from jax import lax
from jax.experimental import pallas as pl
from jax.experimental.pallas import tpu as pltpu
from jax.experimental.pallas import tpu_sc as plsc

N1 = 10000; E1 = 320000; D = 128; B = 8; R = 148
N2 = 1184; E2 = 18944
H = 128; OUT = 2; MH = 1000
EPS = 1e-5

NC = 2
NS = 16
LL = 16
NW = NC * NS
CH = 128


def _sc_scatter_accum(table, src, dst, w, M, width, gather):
    Epad = src.shape[0]
    EperW = Epad // NW
    n_chunks = EperW // CH
    Mr = M // NS
    nf = width // LL
    mesh = plsc.VectorSubcoreMesh(
        core_axis_name="c", subcore_axis_name="s", num_cores=NC, num_subcores=NS)
    zeros = jnp.zeros((M, width), jnp.float32)

    def body(table_hbm, src_hbm, dst_hbm, w_hbm, zeros_hbm, out_hbm,
             src_v, dst_v, w_v, rows_v, sem, agg_sh):
        cid = lax.axis_index("c")
        sid = lax.axis_index("s")
        wid = cid * NS + sid
        @pl.when(sid == 0)
        def _():
            pltpu.sync_copy(zeros_hbm, agg_sh)
        plsc.subcore_barrier()
        base0 = wid * EperW

        def chunk(g, carry):
            base = pl.multiple_of(base0 + g * CH, CH)
            pltpu.sync_copy(dst_hbm.at[pl.ds(base, CH)], dst_v)
            pltpu.sync_copy(w_hbm.at[pl.ds(base, CH)], w_v)
            if gather:
                pltpu.sync_copy(src_hbm.at[pl.ds(base, CH)], src_v)
                pltpu.async_copy(table_hbm.at[src_v], rows_v, sem).wait()

                for j in range(CH // LL):
                    wvec = w_v[pl.ds(j * LL, LL)]
                    for i in range(LL):
                        ws = wvec[i]
                        rv = rows_v.at[j * LL + i]
                        for f in range(nf):
                            rv[pl.ds(f * LL, LL)] = rv[pl.ds(f * LL, LL)] * ws
            else:
                for j in range(CH // LL):
                    wvec = w_v[pl.ds(j * LL, LL)]
                    for i in range(LL):
                        rv = rows_v.at[j * LL + i]
                        for f in range(nf):
                            rv[pl.ds(f * LL, LL)] = (
                                jnp.zeros((LL,), jnp.float32) + wvec[i])
            pltpu.sync_copy(rows_v, agg_sh.at[dst_v], add=True)
            return carry
        lax.fori_loop(0, n_chunks, chunk, 0)
        plsc.subcore_barrier()

        @pl.when(sid == 0)
        def _():
            pltpu.sync_copy(agg_sh, out_hbm.at[cid])

    fn = pl.kernel(
        body,
        out_type=jax.ShapeDtypeStruct((NC, M, width), jnp.float32),
        mesh=mesh,
        scratch_types=[
            pltpu.VMEM((CH,), jnp.int32),
            pltpu.VMEM((CH,), jnp.int32),
            pltpu.VMEM((CH,), jnp.float32),
            pltpu.VMEM((CH, width), jnp.float32),
            pltpu.SemaphoreType.DMA,
            pltpu.VMEM_SHARED((M, width), jnp.float32),
        ],
    )
    return fn(table, src, dst, w, zeros)


def _pad_edges(src, dst, w):
    E = src.shape[0]
    unit = NW * CH
    Epad = ((E + unit - 1) // unit) * unit
    pad = Epad - E
    return (jnp.pad(src, (0, pad)), jnp.pad(dst, (0, pad)),
            jnp.pad(w, (0, pad)))


def _tc_dinv(deg16, M, BN):
    def body(d_ref, o_ref):
        d = d_ref[...]
        s = d[0, :, 0:1] + d[1, :, 0:1] + 1.0
        o_ref[...] = lax.rsqrt(s)
    return pl.pallas_call(
        body, grid=(M // BN,),
        in_specs=[pl.BlockSpec((2, BN, 16), lambda i: (0, i, 0))],
        out_specs=pl.BlockSpec((BN, 1), lambda i: (i, 0)),
        out_shape=jax.ShapeDtypeStruct((M, 1), jnp.float32))(deg16)


def _tc_mm_scale(x, W, dinv, BN):
    N = x.shape[0]
    def body(x_ref, w_ref, d_ref, t_ref, tp_ref):
        t = jnp.dot(x_ref[...], w_ref[...], preferred_element_type=jnp.float32)
        t_ref[...] = t
        tp_ref[...] = t * d_ref[...]
    return pl.pallas_call(
        body, grid=(N // BN,),
        in_specs=[pl.BlockSpec((BN, D), lambda i: (i, 0)),
                  pl.BlockSpec((D, H), lambda i: (0, 0)),
                  pl.BlockSpec((BN, 1), lambda i: (i, 0))],
        out_specs=[pl.BlockSpec((BN, H), lambda i: (i, 0))] * 2,
        out_shape=[jax.ShapeDtypeStruct((N, H), jnp.float32)] * 2)(x, W, dinv)


def _tc_finalize(agg, t, dinv, b, BN):
    N = t.shape[0]
    def body(a_ref, t_ref, d_ref, b_ref, h_ref):
        a = a_ref[...]
        s = a[0] + a[1]
        dv = d_ref[...]
        h = dv * s + dv * dv * t_ref[...] + b_ref[...]
        h_ref[...] = jnp.maximum(h, 0.0)
    return pl.pallas_call(
        body, grid=(N // BN,),
        in_specs=[pl.BlockSpec((2, BN, H), lambda i: (0, i, 0)),
                  pl.BlockSpec((BN, H), lambda i: (i, 0)),
                  pl.BlockSpec((BN, 1), lambda i: (i, 0)),
                  pl.BlockSpec((1, H), lambda i: (0, 0))],
        out_specs=pl.BlockSpec((BN, H), lambda i: (i, 0)),
        out_shape=jax.ShapeDtypeStruct((N, H), jnp.float32))(
            agg, t, dinv, b.reshape(1, H))


def _tc_pool_finalize(sums, cnt16):
    M = sums.shape[1]
    def body(s_ref, c_ref, o_ref):
        s = s_ref[...]
        c = c_ref[...]
        ssum = s[0] + s[1]
        cs = c[0, :, 0:1] + c[1, :, 0:1]
        o_ref[...] = jnp.where(cs > 0, ssum / jnp.maximum(cs, 1.0), 0.0)
    return pl.pallas_call(
        body, grid=(1,),
        in_specs=[pl.BlockSpec((2, M, H), lambda i: (0, 0, 0)),
                  pl.BlockSpec((2, M, 16), lambda i: (0, 0, 0))],
        out_specs=pl.BlockSpec((M, H), lambda i: (0, 0)),
        out_shape=jax.ShapeDtypeStruct((M, H), jnp.float32))(sums, cnt16)


def _tc_mlp(p1, p2, mW1p, mb1p, gp, bep, mW2p, mb2p):
    RB = 4
    NRB = R // RB
    MHP = mW1p.shape[1]
    inv_s = float(1.0 / math.sqrt(1.0 + EPS))

    def body(p1_ref, p2_ref, w1_ref, mb1_ref, g_ref, be_ref, w2_ref, mb2_ref,
             logits_ref, psum_ref, acc_ref):
        rb = pl.program_id(0)
        b = pl.program_id(1)
        ps = p1_ref[0] + p2_ref[0]
        psum_ref[0] = ps
        w1 = w1_ref[...]
        part = jnp.zeros((1, MHP), jnp.float32)
        for i in range(RB):
            part = part + jnp.dot(ps[i:i + 1, :], w1[i * H:(i + 1) * H, :],
                                  preferred_element_type=jnp.float32)

        @pl.when(rb == 0)
        def _():
            acc_ref[pl.ds(b, 1), :] = part

        @pl.when(rb > 0)
        def _():
            acc_ref[pl.ds(b, 1), :] = acc_ref[pl.ds(b, 1), :] + part

        @pl.when(rb == NRB - 1)
        def _():
            z = acc_ref[pl.ds(b, 1), :] + mb1_ref[...]
            z = g_ref[...] * (z * inv_s) + be_ref[...]
            z = jnp.where(z > 0, z, 0.01 * z)
            logits_ref[0] = jnp.dot(z, w2_ref[...],
                                    preferred_element_type=jnp.float32) + mb2_ref[...]

    logits, psum3 = pl.pallas_call(
        body, grid=(NRB, B),
        in_specs=[pl.BlockSpec((1, RB, H), lambda rb, b: (b * NRB + rb, 0, 0)),
                  pl.BlockSpec((1, RB, H), lambda rb, b: (b * NRB + rb, 0, 0)),
                  pl.BlockSpec((RB * H, MHP), lambda rb, b: (rb, 0)),
                  pl.BlockSpec((1, MHP), lambda rb, b: (0, 0)),
                  pl.BlockSpec((1, MHP), lambda rb, b: (0, 0)),
                  pl.BlockSpec((1, MHP), lambda rb, b: (0, 0)),
                  pl.BlockSpec((MHP, 128), lambda rb, b: (0, 0)),
                  pl.BlockSpec((1, 128), lambda rb, b: (0, 0))],
        out_specs=[pl.BlockSpec((1, 1, 128), lambda rb, b: (b, 0, 0)),
                   pl.BlockSpec((1, RB, H), lambda rb, b: (b * NRB + rb, 0, 0))],
        out_shape=[jax.ShapeDtypeStruct((B, 1, 128), jnp.float32),
                   jax.ShapeDtypeStruct((B * R // RB, RB, H), jnp.float32)],
        scratch_shapes=[pltpu.VMEM((B, MHP), jnp.float32)],
    )(p1.reshape(B * R // RB, RB, H), p2.reshape(B * R // RB, RB, H),
      mW1p, mb1p, gp, bep, mW2p, mb2p)
    return logits.reshape(B, 128), psum3.reshape(B * R, H)


def _gcn_branch(x, edge_src, edge_dst, edge_w, Wa, ba, Wb, bb, N, BN):
    deg16 = _sc_scatter_accum(x, edge_src, edge_dst, edge_w, N, 16, False)
    dinv = _tc_dinv(deg16, N, BN)
    t1, tp1 = _tc_mm_scale(x, Wa, dinv, BN)
    agg1 = _sc_scatter_accum(tp1, edge_src, edge_dst, edge_w, N, H, True)
    h1 = _tc_finalize(agg1, t1, dinv, ba, BN)
    t2, tp2 = _tc_mm_scale(h1, Wb, dinv, BN)
    agg2 = _sc_scatter_accum(tp2, edge_src, edge_dst, edge_w, N, H, True)
    h2 = _tc_finalize(agg2, t2, dinv, bb, BN)
    return h2


def _pool(h, seg, N):
    iota = jnp.arange(N, dtype=jnp.int32)
    ones = jnp.ones((N,), jnp.float32)
    s, d, w = _pad_edges(iota, seg, ones)
    cnt16 = _sc_scatter_accum(h, s, d, w, B * R, 16, False)
    sums = _sc_scatter_accum(h, s, d, w, B * R, H, True)
    return _tc_pool_finalize(sums, cnt16)


def kernel(x, node_labels, edge_index, edge_attr, batch, x2, roi_labels,
           edge_index2, edge_attr2, batch2, W1, b1, W2, b2, Wr1, br1, Wr2,
           br2, mW1, mb1, bn_gamma, bn_beta, mW2, mb2):
    i32 = jnp.int32
    ei = edge_index.astype(i32)
    ei2 = edge_index2.astype(i32)
    seg1 = (batch.astype(i32) * R + node_labels.astype(i32))
    seg2 = (batch2.astype(i32) * R + roi_labels.astype(i32))

    s1, d1, w1 = _pad_edges(ei[0], ei[1], edge_attr)
    s2, d2, w2 = _pad_edges(ei2[0], ei2[1], edge_attr2)

    h = _gcn_branch(x, s1, d1, w1, W1, b1, W2, b2, N1, 2000)
    h2 = _gcn_branch(x2, s2, d2, w2, Wr1, br1, Wr2, br2, N2, N2)

    pooled1 = _pool(h, seg1, N1)
    pooled2 = _pool(h2, seg2, N2)

    MHP = 1024
    mW1p = jnp.pad(mW1, ((0, 0), (0, MHP - MH)))
    mb1p = jnp.pad(mb1, (0, MHP - MH)).reshape(1, MHP)
    gp = jnp.pad(bn_gamma, (0, MHP - MH)).reshape(1, MHP)
    bep = jnp.pad(bn_beta, (0, MHP - MH)).reshape(1, MHP)
    mW2p = jnp.pad(mW2, ((0, MHP - MH), (0, 128 - OUT)))
    mb2p = jnp.pad(mb2, (0, 128 - OUT)).reshape(1, 128)

    logits, psum = _tc_mlp(pooled1, pooled2, mW1p, mb1p, gp, bep, mW2p, mb2p)

    out = logits[:, :OUT]
    embedding = pooled1.reshape(B, R * H)
    embedding_roi = pooled2.reshape(B, R * H)
    embedding_sum = psum.reshape(B, R * H)
    return (out, embedding, embedding_roi, embedding_sum)


# speedup vs baseline: 5.3990x; 1.0129x over previous
import functools
import math

import jax
import jax.numpy as jnp
from jax import lax
from jax.experimental import pallas as pl
from jax.experimental.pallas import tpu as pltpu
from jax.experimental.pallas import tpu_sc as plsc

N1 = 10000; E1 = 320000; D = 128; B = 8; R = 148
N2 = 1184; E2 = 18944
H = 128; OUT = 2; MH = 1000
EPS = 1e-5

NC = 2
NS = 16
LL = 16
NW = NC * NS
CH = 128


def _row_group(M):
    for cand in (512, 256, 200, 128, 80, 64, 40, 32, 16, 8):
        if M % cand == 0 and M // cand >= 32:
            return cand
    return 8


def _sc_scatter_accum(table, idxpack, M, width, mode):
    n_chunks = idxpack.shape[0]
    ncpw = n_chunks // NW
    nf = width // LL
    gr = _row_group(M)
    ngr = M // gr
    mesh = plsc.VectorSubcoreMesh(
        core_axis_name="c", subcore_axis_name="s", num_cores=NC, num_subcores=NS)
    zeros = jnp.zeros((M, width), jnp.float32)

    def body(table_hbm, idx_hbm, zeros_hbm, out_hbm,
             ib, dst_v, rows_v, sem, agg_sh):
        cid = lax.axis_index("c")
        sid = lax.axis_index("s")
        wid = cid * NS + sid
        @pl.when(sid == 0)
        def _():
            pltpu.sync_copy(zeros_hbm, agg_sh)
        plsc.subcore_barrier()

        def chunk(c, carry):
            pltpu.sync_copy(idx_hbm.at[wid * ncpw + c], ib)
            pltpu.sync_copy(idx_hbm.at[wid * ncpw + c, 1], dst_v)
            if mode in ('gather_scale', 'gather'):
                pltpu.async_copy(table_hbm.at[ib.at[0]], rows_v, sem).wait()
            wrow = ib.at[2]
            if mode == 'gather_scale':
                for j in range(CH // LL):
                    wvec = lax.bitcast_convert_type(wrow[pl.ds(j * LL, LL)], jnp.float32)
                    for i in range(LL):
                        ws = wvec[i]
                        rv = rows_v.at[j * LL + i]
                        for f in range(nf):
                            rv[pl.ds(f * LL, LL)] = rv[pl.ds(f * LL, LL)] * ws
            elif mode == 'fill':
                for j in range(CH // LL):
                    wvec = lax.bitcast_convert_type(wrow[pl.ds(j * LL, LL)], jnp.float32)
                    for i in range(LL):
                        rv = rows_v.at[j * LL + i]
                        for f in range(nf):
                            rv[pl.ds(f * LL, LL)] = (
                                jnp.zeros((LL,), jnp.float32) + wvec[i])
            pltpu.sync_copy(rows_v, agg_sh.at[dst_v], add=True)
            return carry
        lax.fori_loop(0, ncpw, chunk, 0)
        plsc.subcore_barrier()

        @pl.when(sid == 0)
        def _():
            pltpu.sync_copy(agg_sh, out_hbm.at[cid])

    fn = pl.kernel(
        body,
        out_type=jax.ShapeDtypeStruct((NC, M, width), jnp.float32),
        mesh=mesh,
        scratch_types=[
            pltpu.VMEM((3, CH), jnp.int32),
            pltpu.VMEM((CH,), jnp.int32),
            pltpu.VMEM((CH, width), jnp.float32),
            pltpu.SemaphoreType.DMA,
            pltpu.VMEM_SHARED((M, width), jnp.float32),
        ],
    )
    return fn(table, idxpack, zeros)


def _pack_edges(src, dst, w, M, src_pad):
    E = src.shape[0]
    unit = NW * CH
    Epad = ((E + unit - 1) // unit) * unit
    pad = Epad - E
    srcp = jnp.concatenate([src, jnp.full((pad,), src_pad, jnp.int32)])
    dstp = jnp.concatenate([dst, jnp.arange(pad, dtype=jnp.int32) % M])
    wp = jnp.pad(w, (0, pad))
    w32 = lax.bitcast_convert_type(wp, jnp.int32)
    return jnp.stack([srcp.reshape(-1, CH), dstp.reshape(-1, CH),
                      w32.reshape(-1, CH)], axis=1)


def _tc_dinv(deg16, M, BN):
    def body(d_ref, o_ref):
        d = d_ref[...]
        s = d[0, :, 0:1] + d[1, :, 0:1] + 1.0
        o_ref[...] = lax.rsqrt(s)
    return pl.pallas_call(
        body, grid=(M // BN,),
        in_specs=[pl.BlockSpec((2, BN, 16), lambda i: (0, i, 0))],
        out_specs=pl.BlockSpec((BN, 1), lambda i: (i, 0)),
        out_shape=jax.ShapeDtypeStruct((M, 1), jnp.float32))(deg16)


def _tc_mm_scale(x, W, dinv, BN):
    N = x.shape[0]
    def body(x_ref, w_ref, d_ref, t_ref, tp_ref):
        t = jnp.dot(x_ref[...], w_ref[...], preferred_element_type=jnp.float32)
        t_ref[...] = t
        tp_ref[...] = t * d_ref[...]
    return pl.pallas_call(
        body, grid=(N // BN,),
        in_specs=[pl.BlockSpec((BN, D), lambda i: (i, 0)),
                  pl.BlockSpec((D, H), lambda i: (0, 0)),
                  pl.BlockSpec((BN, 1), lambda i: (i, 0))],
        out_specs=[pl.BlockSpec((BN, H), lambda i: (i, 0))] * 2,
        out_shape=[jax.ShapeDtypeStruct((N, H), jnp.float32)] * 2)(x, W, dinv)


def _tc_finalize(agg, t, dinv, b, BN):
    N = t.shape[0]
    def body(a_ref, t_ref, d_ref, b_ref, h_ref):
        a = a_ref[...]
        s = a[0] + a[1]
        dv = d_ref[...]
        h = dv * s + dv * dv * t_ref[...] + b_ref[...]
        h_ref[...] = jnp.maximum(h, 0.0)
    return pl.pallas_call(
        body, grid=(N // BN,),
        in_specs=[pl.BlockSpec((2, BN, H), lambda i: (0, i, 0)),
                  pl.BlockSpec((BN, H), lambda i: (i, 0)),
                  pl.BlockSpec((BN, 1), lambda i: (i, 0)),
                  pl.BlockSpec((1, H), lambda i: (0, 0))],
        out_specs=pl.BlockSpec((BN, H), lambda i: (i, 0)),
        out_shape=jax.ShapeDtypeStruct((N, H), jnp.float32))(
            agg, t, dinv, b.reshape(1, H))


def _tc_pool_finalize(sums, cnt16):
    M = sums.shape[1]
    def body(s_ref, c_ref, o_ref):
        s = s_ref[...]
        c = c_ref[...]
        ssum = s[0] + s[1]
        cs = c[0, :, 0:1] + c[1, :, 0:1]
        o_ref[...] = jnp.where(cs > 0, ssum / jnp.maximum(cs, 1.0), 0.0)
    return pl.pallas_call(
        body, grid=(1,),
        in_specs=[pl.BlockSpec((2, M, H), lambda i: (0, 0, 0)),
                  pl.BlockSpec((2, M, 16), lambda i: (0, 0, 0))],
        out_specs=pl.BlockSpec((M, H), lambda i: (0, 0)),
        out_shape=jax.ShapeDtypeStruct((M, H), jnp.float32))(sums, cnt16)


def _tc_mlp(p1, p2, mW1p, mb1p, gp, bep, mW2p, mb2p):
    RB = 4
    NRB = R // RB
    MHP = mW1p.shape[1]
    inv_s = float(1.0 / math.sqrt(1.0 + EPS))

    def body(p1_ref, p2_ref, w1_ref, mb1_ref, g_ref, be_ref, w2_ref, mb2_ref,
             logits_ref, psum_ref, acc_ref):
        rb = pl.program_id(0)
        b = pl.program_id(1)
        ps = p1_ref[0] + p2_ref[0]
        psum_ref[0] = ps
        w1 = w1_ref[...]
        part = jnp.zeros((1, MHP), jnp.float32)
        for i in range(RB):
            part = part + jnp.dot(ps[i:i + 1, :], w1[i * H:(i + 1) * H, :],
                                  preferred_element_type=jnp.float32)

        @pl.when(rb == 0)
        def _():
            acc_ref[pl.ds(b, 1), :] = part

        @pl.when(rb > 0)
        def _():
            acc_ref[pl.ds(b, 1), :] = acc_ref[pl.ds(b, 1), :] + part

        @pl.when(rb == NRB - 1)
        def _():
            z = acc_ref[pl.ds(b, 1), :] + mb1_ref[...]
            z = g_ref[...] * (z * inv_s) + be_ref[...]
            z = jnp.where(z > 0, z, 0.01 * z)
            logits_ref[0] = jnp.dot(z, w2_ref[...],
                                    preferred_element_type=jnp.float32) + mb2_ref[...]

    logits, psum3 = pl.pallas_call(
        body, grid=(NRB, B),
        in_specs=[pl.BlockSpec((1, RB, H), lambda rb, b: (b * NRB + rb, 0, 0)),
                  pl.BlockSpec((1, RB, H), lambda rb, b: (b * NRB + rb, 0, 0)),
                  pl.BlockSpec((RB * H, MHP), lambda rb, b: (rb, 0)),
                  pl.BlockSpec((1, MHP), lambda rb, b: (0, 0)),
                  pl.BlockSpec((1, MHP), lambda rb, b: (0, 0)),
                  pl.BlockSpec((1, MHP), lambda rb, b: (0, 0)),
                  pl.BlockSpec((MHP, 128), lambda rb, b: (0, 0)),
                  pl.BlockSpec((1, 128), lambda rb, b: (0, 0))],
        out_specs=[pl.BlockSpec((1, 1, 128), lambda rb, b: (b, 0, 0)),
                   pl.BlockSpec((1, RB, H), lambda rb, b: (b * NRB + rb, 0, 0))],
        out_shape=[jax.ShapeDtypeStruct((B, 1, 128), jnp.float32),
                   jax.ShapeDtypeStruct((B * R // RB, RB, H), jnp.float32)],
        scratch_shapes=[pltpu.VMEM((B, MHP), jnp.float32)],
    )(p1.reshape(B * R // RB, RB, H), p2.reshape(B * R // RB, RB, H),
      mW1p, mb1p, gp, bep, mW2p, mb2p)
    return logits.reshape(B, 128), psum3.reshape(B * R, H)


def _gcn_branch(x, epack, Wa, ba, Wb, bb, N, BN):
    deg16 = _sc_scatter_accum(x, epack, N, 16, 'fill')
    dinv = _tc_dinv(deg16, N, BN)
    t1, tp1 = _tc_mm_scale(x, Wa, dinv, BN)
    agg1 = _sc_scatter_accum(tp1, epack, N, H, 'gather_scale')
    h1 = _tc_finalize(agg1, t1, dinv, ba, BN)
    t2, tp2 = _tc_mm_scale(h1, Wb, dinv, BN)
    agg2 = _sc_scatter_accum(tp2, epack, N, H, 'gather_scale')
    h2 = _tc_finalize(agg2, t2, dinv, bb, BN)
    return h2


def _pool(h, seg, N):
    iota = jnp.arange(N, dtype=jnp.int32)
    ones = jnp.ones((N,), jnp.float32)
    ppack = _pack_edges(iota, seg, ones, B * R, N)
    hpad = jnp.pad(h, ((0, 8), (0, 0)))
    cnt16 = _sc_scatter_accum(hpad, ppack, B * R, 16, 'fill')
    sums = _sc_scatter_accum(hpad, ppack, B * R, H, 'gather')
    return _tc_pool_finalize(sums, cnt16)


def kernel(x, node_labels, edge_index, edge_attr, batch, x2, roi_labels,
           edge_index2, edge_attr2, batch2, W1, b1, W2, b2, Wr1, br1, Wr2,
           br2, mW1, mb1, bn_gamma, bn_beta, mW2, mb2):
    i32 = jnp.int32
    ei = edge_index.astype(i32)
    ei2 = edge_index2.astype(i32)
    seg1 = (batch.astype(i32) * R + node_labels.astype(i32))
    seg2 = (batch2.astype(i32) * R + roi_labels.astype(i32))

    epack1 = _pack_edges(ei[0], ei[1], edge_attr, N1, 0)
    epack2 = _pack_edges(ei2[0], ei2[1], edge_attr2, N2, 0)

    h = _gcn_branch(x, epack1, W1, b1, W2, b2, N1, 2000)
    h2 = _gcn_branch(x2, epack2, Wr1, br1, Wr2, br2, N2, N2)

    pooled1 = _pool(h, seg1, N1)
    pooled2 = _pool(h2, seg2, N2)

    MHP = 1024
    mW1p = jnp.pad(mW1, ((0, 0), (0, MHP - MH)))
    mb1p = jnp.pad(mb1, (0, MHP - MH)).reshape(1, MHP)
    gp = jnp.pad(bn_gamma, (0, MHP - MH)).reshape(1, MHP)
    bep = jnp.pad(bn_beta, (0, MHP - MH)).reshape(1, MHP)
    mW2p = jnp.pad(mW2, ((0, MHP - MH), (0, 128 - OUT)))
    mb2p = jnp.pad(mb2, (0, 128 - OUT)).reshape(1, 128)

    logits, psum = _tc_mlp(pooled1, pooled2, mW1p, mb1p, gp, bep, mW2p, mb2p)

    out = logits[:, :OUT]
    embedding = pooled1.reshape(B, R * H)
    embedding_roi = pooled2.reshape(B, R * H)
    embedding_sum = psum.reshape(B, R * H)
    return (out, embedding, embedding_roi, embedding_sum)
